# Initial kernel scaffold; baseline (speedup 1.0000x reference)
import functools

import jax
import jax.numpy as jnp
from jax import lax
from jax.experimental import pallas as pl
from jax.experimental.pallas import tpu as pltpu
from jax.experimental.pallas import tpu_sc as plsc

NUM_BUCKETS = 2048
HALF_DIM = 32
EMBEDDING_DIM = 64

_info = plsc.get_sparse_core_info()
_NC, _NS, _L = _info.num_cores, _info.num_subcores, _info.num_lanes
_NW = _NC * _NS

_IDX_COLS = 128
_ROWS_PER_CHUNK = 8
_CHUNK = _IDX_COLS * _ROWS_PER_CHUNK


def _proj_body(t_ref, w_ref, b_ref, o_ref):
    o_ref[...] = (
        jnp.dot(t_ref[...], w_ref[...], preferred_element_type=jnp.float32)
        + b_ref[...]
    )


def _project_table(bucket_table, W, b):
    return pl.pallas_call(
        _proj_body,
        out_shape=jax.ShapeDtypeStruct((NUM_BUCKETS, EMBEDDING_DIM), jnp.float32),
    )(bucket_table, W, b.reshape(1, EMBEDDING_DIM))


def _make_gather(total_rows):
    assert total_rows % (_NW * _CHUNK) == 0
    chunks_per_worker = total_rows // (_NW * _CHUNK)
    rows_per_worker = chunks_per_worker * _CHUNK
    idx_rows_per_worker = rows_per_worker // _IDX_COLS
    mesh = plsc.VectorSubcoreMesh(core_axis_name="c", subcore_axis_name="s")

    @functools.partial(
        pl.kernel,
        out_type=jax.ShapeDtypeStruct((total_rows, EMBEDDING_DIM), jnp.float32),
        mesh=mesh,
        scratch_types=[
            pltpu.VMEM((_ROWS_PER_CHUNK, _IDX_COLS), jnp.int32),
            pltpu.VMEM((_CHUNK, EMBEDDING_DIM), jnp.float32),
            pltpu.SemaphoreType.DMA,
        ],
    )
    def gather_kernel(p_hbm, x_hbm, out_hbm, idx_v, rows_v, sem):
        wid = lax.axis_index("s") * _NC + lax.axis_index("c")
        idx_row0 = wid * idx_rows_per_worker
        out_row0 = wid * rows_per_worker

        def chunk_body(c, carry):
            pltpu.sync_copy(
                x_hbm.at[pl.ds(idx_row0 + c * _ROWS_PER_CHUNK, _ROWS_PER_CHUNK)],
                idx_v,
            )
            for r in range(_ROWS_PER_CHUNK):
                for g in range(_IDX_COLS // _L):
                    sl = pl.ds(g * _L, _L)
                    idx_v[r, sl] = lax.bitwise_and(idx_v[r, sl], NUM_BUCKETS - 1)
            copies = []
            for r in range(_ROWS_PER_CHUNK):
                copies.append(
                    pltpu.async_copy(
                        p_hbm.at[idx_v.at[r]],
                        rows_v.at[pl.ds(r * _IDX_COLS, _IDX_COLS)],
                        sem,
                    )
                )
            for cp in copies:
                cp.wait()
            pltpu.sync_copy(
                rows_v, out_hbm.at[pl.ds(out_row0 + c * _CHUNK, _CHUNK)]
            )
            return carry

        lax.fori_loop(0, chunks_per_worker, chunk_body, 0)

    return gather_kernel


def kernel(x, bucket_table, W, b):
    B, L = x.shape
    total = B * L
    x2 = x.reshape(total // _IDX_COLS, _IDX_COLS).astype(jnp.int32)
    P = _project_table(bucket_table, W, b)
    out = _make_gather(total)(P, x2)
    return out.reshape(B, L, EMBEDDING_DIM)


# speedup vs baseline: 3.9686x; 3.9686x over previous
import functools

import jax
import jax.numpy as jnp
from jax import lax
from jax.experimental import pallas as pl
from jax.experimental.pallas import tpu as pltpu
from jax.experimental.pallas import tpu_sc as plsc

NUM_BUCKETS = 2048
HALF_DIM = 32
EMBEDDING_DIM = 64

_info = plsc.get_sparse_core_info()
_NC, _NS, _L = _info.num_cores, _info.num_subcores, _info.num_lanes
_NW = _NC * _NS

_IDX_COLS = 128
_ROWS_PER_CHUNK = 8
_CHUNK = _IDX_COLS * _ROWS_PER_CHUNK


def _proj_body(t_ref, w_ref, b_ref, o_ref):
    o_ref[...] = (
        jnp.dot(t_ref[...], w_ref[...], preferred_element_type=jnp.float32)
        + b_ref[...]
    )


def _project_table(bucket_table, W, b):
    return pl.pallas_call(
        _proj_body,
        out_shape=jax.ShapeDtypeStruct((NUM_BUCKETS, EMBEDDING_DIM), jnp.float32),
    )(bucket_table, W, b.reshape(1, EMBEDDING_DIM))


def _make_gather(total_rows):
    assert total_rows % (_NW * _CHUNK) == 0
    chunks_per_worker = total_rows // (_NW * _CHUNK)
    rows_per_worker = chunks_per_worker * _CHUNK
    idx_rows_per_worker = rows_per_worker // _IDX_COLS
    mesh = plsc.VectorSubcoreMesh(core_axis_name="c", subcore_axis_name="s")

    @functools.partial(
        pl.kernel,
        out_type=jax.ShapeDtypeStruct((total_rows, EMBEDDING_DIM), jnp.float32),
        mesh=mesh,
        scratch_types=[
            pltpu.VMEM((_ROWS_PER_CHUNK, _IDX_COLS), jnp.int32),
            pltpu.VMEM((_CHUNK, EMBEDDING_DIM), jnp.float32),
            pltpu.SemaphoreType.DMA,
        ],
        compiler_params=pltpu.CompilerParams(use_tc_tiling_on_sc=False),
    )
    def gather_kernel(p_hbm, x_hbm, out_hbm, idx_v, rows_v, sem):
        wid = lax.axis_index("s") * _NC + lax.axis_index("c")
        idx_row0 = wid * idx_rows_per_worker
        out_row0 = wid * rows_per_worker

        def chunk_body(c, carry):
            pltpu.sync_copy(
                x_hbm.at[pl.ds(idx_row0 + c * _ROWS_PER_CHUNK, _ROWS_PER_CHUNK)],
                idx_v,
            )
            for r in range(_ROWS_PER_CHUNK):
                for g in range(_IDX_COLS // _L):
                    sl = pl.ds(g * _L, _L)
                    idx_v[r, sl] = lax.bitwise_and(idx_v[r, sl], NUM_BUCKETS - 1)
            copies = []
            for r in range(_ROWS_PER_CHUNK):
                copies.append(
                    pltpu.async_copy(
                        p_hbm.at[idx_v.at[r]],
                        rows_v.at[pl.ds(r * _IDX_COLS, _IDX_COLS)],
                        sem,
                    )
                )
            for cp in copies:
                cp.wait()
            pltpu.sync_copy(
                rows_v, out_hbm.at[pl.ds(out_row0 + c * _CHUNK, _CHUNK)]
            )
            return carry

        lax.fori_loop(0, chunks_per_worker, chunk_body, 0)

    return gather_kernel


def kernel(x, bucket_table, W, b):
    B, L = x.shape
    total = B * L
    x2 = x.reshape(total // _IDX_COLS, _IDX_COLS).astype(jnp.int32)
    P = _project_table(bucket_table, W, b)
    out = _make_gather(total)(P, x2)
    return out.reshape(B, L, EMBEDDING_DIM)
